# baseline (speedup 1.0000x reference)
import functools

import jax
import jax.numpy as jnp
from jax import lax
from jax.experimental import pallas as pl
from jax.experimental.pallas import tpu as pltpu
from jax.experimental.pallas import tpu_sc as plsc

S, B, E = 8192, 4, 1024
L = 16
NC, NS = 2, 16
NW = NC * NS
RW = S // NW
CH = 2
NCHUNK = RW // CH
NSLOT = 4


@functools.partial(
    pl.kernel,
    out_type=jax.ShapeDtypeStruct((S, B, E), jnp.float32),
    mesh=plsc.VectorSubcoreMesh(core_axis_name="c", subcore_axis_name="s"),
    scratch_types=(
        [pltpu.VMEM((CH, B, E), jnp.float32) for _ in range(NSLOT)]
        + [pltpu.VMEM((CH, E), jnp.float32) for _ in range(NSLOT)]
        + [pltpu.VMEM((CH, B, E), jnp.float32) for _ in range(NSLOT)]
        + [pltpu.SemaphoreType.DMA for _ in range(3 * NSLOT)]
    ),
)
def _sc_add(in_hbm, tab_hbm, out_hbm, *refs):
    in_bufs = refs[0:NSLOT]
    tab_bufs = refs[NSLOT:2 * NSLOT]
    out_bufs = refs[2 * NSLOT:3 * NSLOT]
    in_sems = refs[3 * NSLOT:4 * NSLOT]
    tab_sems = refs[4 * NSLOT:5 * NSLOT]
    out_sems = refs[5 * NSLOT:6 * NSLOT]

    wid = lax.axis_index("s") * NC + lax.axis_index("c")
    base0 = wid * RW

    def start_in(c, p):
        row = base0 + c * CH
        pltpu.make_async_copy(in_hbm.at[pl.ds(row, CH)], in_bufs[p], in_sems[p]).start()
        pltpu.make_async_copy(tab_hbm.at[pl.ds(row, CH)], tab_bufs[p], tab_sems[p]).start()

    for p in range(NSLOT):
        start_in(p, p)

    def outer(c0, carry):
        for p in range(NSLOT):
            c = c0 * NSLOT + p
            pltpu.make_async_copy(in_hbm.at[pl.ds(0, CH)], in_bufs[p], in_sems[p]).wait()
            pltpu.make_async_copy(tab_hbm.at[pl.ds(0, CH)], tab_bufs[p], tab_sems[p]).wait()

            @pl.when(c0 > 0)
            def _wait_prev_out(p=p):
                pltpu.make_async_copy(out_bufs[p], out_hbm.at[pl.ds(0, CH)], out_sems[p]).wait()

            def slab(t, cy, p=p):
                r = t // (E // L)
                j = (t % (E // L)) * L
                tab = tab_bufs[p][r, pl.ds(j, L)]
                for b in range(B):
                    out_bufs[p][r, b, pl.ds(j, L)] = in_bufs[p][r, b, pl.ds(j, L)] + tab
                return cy

            lax.fori_loop(0, CH * (E // L), slab, 0)

            row = base0 + c * CH
            pltpu.make_async_copy(out_bufs[p], out_hbm.at[pl.ds(row, CH)], out_sems[p]).start()

            @pl.when(c0 < NCHUNK // NSLOT - 1)
            def _start_next_in(c=c, p=p):
                start_in(c + NSLOT, p)

        return carry

    lax.fori_loop(0, NCHUNK // NSLOT, outer, 0)

    for p in range(NSLOT):
        pltpu.make_async_copy(out_bufs[p], out_hbm.at[pl.ds(0, CH)], out_sems[p]).wait()


def kernel(input, pos_table):
    return _sc_add(input, pos_table)
